# Initial kernel scaffold; baseline (speedup 1.0000x reference)
import jax
import jax.numpy as jnp
from jax.experimental import pallas as pl


def kernel(x_in, edge_index, edge_weight, idx, W1, b1, W2, b2, W3, b3, W4, b4):
    raise NotImplementedError("write your pallas kernel here")


# speedup vs baseline: 3.7809x; 3.7809x over previous
import functools

import jax
import jax.numpy as jnp
from jax import lax
from jax.experimental import pallas as pl
from jax.experimental.pallas import tpu as pltpu
from jax.experimental.pallas import tpu_sc as plsc

N_NODES = 10000
N_PAD = 10240
DIM = 128
NUM_CLASSES = 32
NUM_GRAPHS = 64
NC = 2
NS = 16
NW = NC * NS
LANES = 16
CHUNK = 128
ROWS_PER_TILE = N_PAD // NS


def _spmm_sc(x, col_p, row_p, w_p):
    e_pad = col_p.shape[0]
    per_w = e_pad // NW
    n_chunks = per_w // CHUNK
    mesh = plsc.VectorSubcoreMesh(core_axis_name="c", subcore_axis_name="s")

    @functools.partial(
        pl.kernel,
        out_type=jax.ShapeDtypeStruct((NC, N_PAD, DIM), jnp.float32),
        mesh=mesh,
        scratch_types=[
            pltpu.VMEM((CHUNK,), jnp.int32),
            pltpu.VMEM((CHUNK,), jnp.int32),
            pltpu.VMEM((CHUNK,), jnp.float32),
            pltpu.VMEM((CHUNK, DIM), jnp.float32),
            pltpu.VMEM_SHARED((N_PAD, DIM), jnp.float32),
            pltpu.SemaphoreType.DMA,
        ],
    )
    def spmm_kernel(x_hbm, col_hbm, row_hbm, w_hbm, out_hbm,
                    colv, rowv, wv, rows, acc, sem):
        c = lax.axis_index("c")
        s = lax.axis_index("s")
        wid = c * NS + s
        base = wid * per_w

        def zero_body(i, _):
            rows[i // 8, pl.ds((i % 8) * LANES, LANES)] = jnp.zeros(
                (LANES,), jnp.float32)
            return 0
        lax.fori_loop(0, CHUNK * 8, zero_body, 0)

        def zcopy_body(k, _):
            pltpu.sync_copy(
                rows, acc.at[pl.ds(s * ROWS_PER_TILE + k * CHUNK, CHUNK)])
            return 0
        lax.fori_loop(0, ROWS_PER_TILE // CHUNK, zcopy_body, 0)
        plsc.subcore_barrier()

        def chunk_body(ci, _):
            off = base + ci * CHUNK
            pltpu.sync_copy(col_hbm.at[pl.ds(off, CHUNK)], colv)
            pltpu.async_copy(x_hbm.at[colv], rows, sem).wait()
            pltpu.sync_copy(w_hbm.at[pl.ds(off, CHUNK)], wv)
            pltpu.sync_copy(row_hbm.at[pl.ds(off, CHUNK)], rowv)

            def group_body(g, _):
                w16 = wv[pl.ds(g * LANES, LANES)]
                for jj in range(LANES):
                    wj = w16[jj]
                    for f in range(DIM // LANES):
                        sl = pl.ds(f * LANES, LANES)
                        rows[g * LANES + jj, sl] = rows[g * LANES + jj, sl] * wj
                return 0
            lax.fori_loop(0, CHUNK // LANES, group_body, 0)
            pltpu.sync_copy(rows, acc.at[rowv], add=True)
            return 0
        lax.fori_loop(0, n_chunks, chunk_body, 0)
        plsc.subcore_barrier()

        def rb_body(k, _):
            r0 = s * ROWS_PER_TILE + k * CHUNK
            pltpu.sync_copy(acc.at[pl.ds(r0, CHUNK)], rows)
            pltpu.sync_copy(rows, out_hbm.at[c, pl.ds(r0, CHUNK)])
            return 0
        lax.fori_loop(0, ROWS_PER_TILE // CHUNK, rb_body, 0)

    return spmm_kernel(x, col_p, row_p, w_p)


_BR = 1024


def _dense_relu_tc(y, w, b):
    def body(y_ref, w_ref, b_ref, o_ref):
        ysum = y_ref[0] + y_ref[1]
        acc = lax.dot_general(ysum, w_ref[...], (((1,), (1,)), ((), ())),
                              preferred_element_type=jnp.float32)
        o_ref[...] = jnp.maximum(acc + b_ref[...], 0.0)

    return pl.pallas_call(
        body,
        grid=(N_PAD // _BR,),
        in_specs=[
            pl.BlockSpec((NC, _BR, DIM), lambda i: (0, i, 0)),
            pl.BlockSpec((DIM, DIM), lambda i: (0, 0)),
            pl.BlockSpec((1, DIM), lambda i: (0, 0)),
        ],
        out_specs=pl.BlockSpec((_BR, DIM), lambda i: (i, 0)),
        out_shape=jax.ShapeDtypeStruct((N_PAD, DIM), jnp.float32),
    )(y, w, b.reshape(1, DIM))


def _final_tc(z, idx3, w2, b2, w3, b3, w4, b4):
    nb = N_PAD // _BR

    def body(z_ref, idx_ref, w2_ref, b2_ref, w3_ref, b3_ref, w4_ref, b4_ref,
             o_ref, pool_ref):
        i = pl.program_id(0)

        @pl.when(i == 0)
        def _():
            pool_ref[...] = jnp.zeros((NUM_GRAPHS, DIM), jnp.float32)

        zsum = z_ref[0] + z_ref[1]
        x2 = lax.dot_general(zsum, w2_ref[...], (((1,), (1,)), ((), ())),
                             preferred_element_type=jnp.float32)
        x2 = jnp.maximum(x2 + b2_ref[...], 0.0)

        gids = lax.broadcasted_iota(jnp.int32, (NUM_GRAPHS, _BR), 0)
        onehot = (gids == idx_ref[0]).astype(jnp.float32)
        pool_ref[...] += lax.dot_general(
            onehot, x2, (((1,), (0,)), ((), ())),
            preferred_element_type=jnp.float32)

        @pl.when(i == nb - 1)
        def _():
            h = lax.dot_general(pool_ref[...], w3_ref[...],
                                (((1,), (1,)), ((), ())),
                                preferred_element_type=jnp.float32)
            h = jnp.maximum(h + b3_ref[...], 0.0)
            logits = lax.dot_general(h, w4_ref[...], (((1,), (1,)), ((), ())),
                                     preferred_element_type=jnp.float32)
            logits = logits + b4_ref[...]
            m = jnp.max(logits, axis=1, keepdims=True)
            lse = m + jnp.log(jnp.sum(jnp.exp(logits - m), axis=1,
                                      keepdims=True))
            o_ref[...] = logits - lse

    return pl.pallas_call(
        body,
        grid=(nb,),
        in_specs=[
            pl.BlockSpec((NC, _BR, DIM), lambda i: (0, i, 0)),
            pl.BlockSpec((1, 1, _BR), lambda i: (i, 0, 0)),
            pl.BlockSpec((DIM, DIM), lambda i: (0, 0)),
            pl.BlockSpec((1, DIM), lambda i: (0, 0)),
            pl.BlockSpec((DIM, DIM), lambda i: (0, 0)),
            pl.BlockSpec((1, DIM), lambda i: (0, 0)),
            pl.BlockSpec((NUM_CLASSES, DIM), lambda i: (0, 0)),
            pl.BlockSpec((1, NUM_CLASSES), lambda i: (0, 0)),
        ],
        out_specs=pl.BlockSpec((NUM_GRAPHS, NUM_CLASSES), lambda i: (0, 0)),
        out_shape=jax.ShapeDtypeStruct((NUM_GRAPHS, NUM_CLASSES), jnp.float32),
        scratch_shapes=[pltpu.VMEM((NUM_GRAPHS, DIM), jnp.float32)],
    )(z, idx3, w2, b2.reshape(1, DIM), w3, b3.reshape(1, DIM), w4,
      b4.reshape(1, NUM_CLASSES))


def kernel(x_in, edge_index, edge_weight, idx, W1, b1, W2, b2, W3, b3, W4, b4):
    e = edge_index.shape[1]
    e_pad = ((e + NW * CHUNK - 1) // (NW * CHUNK)) * (NW * CHUNK)
    pad = e_pad - e
    row = edge_index[0].astype(jnp.int32)
    col = edge_index[1].astype(jnp.int32)
    if pad:
        row = jnp.concatenate([row, jnp.zeros((pad,), jnp.int32)])
        col = jnp.concatenate([col, jnp.zeros((pad,), jnp.int32)])
        w = jnp.concatenate([edge_weight, jnp.zeros((pad,), jnp.float32)])
    else:
        w = edge_weight

    y = _spmm_sc(x_in, col, row, w)
    x1 = _dense_relu_tc(y, W1, b1)
    z = _spmm_sc(x1, col, row, w)
    idxp = jnp.concatenate([idx.astype(jnp.int32),
                            jnp.full((N_PAD - N_NODES,), NUM_GRAPHS, jnp.int32)])
    idx3 = idxp.reshape(N_PAD // _BR, 1, _BR)
    return _final_tc(z, idx3, W2, b2, W3, b3, W4, b4)


# speedup vs baseline: 6.5754x; 1.7391x over previous
import functools

import jax
import jax.numpy as jnp
from jax import lax
from jax.experimental import pallas as pl
from jax.experimental.pallas import tpu as pltpu
from jax.experimental.pallas import tpu_sc as plsc

N_NODES = 10000
N_PAD = 10240
DIM = 128
NUM_CLASSES = 32
NUM_GRAPHS = 64
NC = 2
NS = 16
NW = NC * NS
LANES = 16
CHUNK = 112
NSLOTS = 3
CHUNKS_PER_W = 90
E_PAD = NW * CHUNKS_PER_W * CHUNK
ROWS_PER_TILE = N_PAD // NS
RB = 80


def _spmm_sc(x, packed, wblk):
    n_triples = CHUNKS_PER_W // NSLOTS
    mesh = plsc.VectorSubcoreMesh(core_axis_name="c", subcore_axis_name="s")

    @functools.partial(
        pl.kernel,
        out_type=jax.ShapeDtypeStruct((NC, N_PAD, DIM), jnp.float32),
        mesh=mesh,
        scratch_types=[
            pltpu.VMEM((NSLOTS, 2, CHUNK), jnp.int32),
            pltpu.VMEM((NSLOTS, CHUNK), jnp.float32),
            pltpu.VMEM((NSLOTS, CHUNK, DIM), jnp.float32),
            pltpu.VMEM_SHARED((N_PAD, DIM), jnp.float32),
            [pltpu.SemaphoreType.DMA] * NSLOTS,
            [pltpu.SemaphoreType.DMA] * NSLOTS,
            [pltpu.SemaphoreType.DMA] * NSLOTS,
            [pltpu.SemaphoreType.DMA] * NSLOTS,
        ],
    )
    def spmm_kernel(x_hbm, pk_hbm, w_hbm, out_hbm, idxs, wvs, rowss, acc,
                    sems_i, sems_w, sems_g, sems_s):
        c = lax.axis_index("c")
        s = lax.axis_index("s")
        wid = c * NS + s
        cbase = wid * CHUNKS_PER_W
        idx = [idxs.at[k] for k in range(NSLOTS)]
        rows = [rowss.at[k] for k in range(NSLOTS)]

        def zero_body(i, _):
            rowss[0, i // 8, pl.ds((i % 8) * LANES, LANES)] = jnp.zeros(
                (LANES,), jnp.float32)
            return 0
        lax.fori_loop(0, RB * 8, zero_body, 0)

        def zcopy_body(k, _):
            pltpu.sync_copy(rows[0].at[pl.ds(0, RB)],
                            acc.at[pl.ds(s * ROWS_PER_TILE + k * RB, RB)])
            return 0
        lax.fori_loop(0, ROWS_PER_TILE // RB, zcopy_body, 0)
        plsc.subcore_barrier()

        def issue_idx(blk, k):
            pltpu.async_copy(pk_hbm.at[blk], idx[k], sems_i[k])
            pltpu.async_copy(w_hbm.at[blk], wvs.at[k], sems_w[k])

        def wait_idx(blk, k):
            pltpu.make_async_copy(pk_hbm.at[blk], idx[k], sems_i[k]).wait()
            pltpu.make_async_copy(w_hbm.at[blk], wvs.at[k], sems_w[k]).wait()

        def issue_gather(k):
            return pltpu.async_copy(x_hbm.at[idx[k].at[0]], rows[k],
                                    sems_g[k])

        def wait_gather(k):
            pltpu.make_async_copy(x_hbm.at[idx[k].at[0]], rows[k],
                                  sems_g[k]).wait()

        def issue_scatter(k):
            return pltpu.async_copy(rows[k], acc.at[idx[k].at[1]], sems_s[k],
                                    add=True)

        def wait_scatter(k):
            pltpu.make_async_copy(rows[k], acc.at[idx[k].at[1]],
                                  sems_s[k]).wait()

        def scale(k):
            def group_body(g, _):
                w16 = wvs[k, pl.ds(g * LANES, LANES)]
                for jj in range(LANES):
                    wj = w16[jj]
                    r = g * LANES + jj
                    for f in range(DIM // LANES):
                        sl = pl.ds(f * LANES, LANES)
                        rowss[k, r, sl] = rowss[k, r, sl] * wj
                return 0
            lax.fori_loop(0, CHUNK // LANES, group_body, 0)

        issue_idx(cbase, 0)
        issue_idx(cbase + 1, 1)
        issue_idx(cbase + 2, 2)
        wait_idx(cbase, 0)
        issue_gather(0)

        def triple_body(t, _):
            c0 = cbase + 3 * t

            wait_gather(0)
            wait_idx(c0 + 1, 1)
            issue_gather(1)
            scale(0)
            issue_scatter(0)

            @pl.when(t > 0)
            def _():
                wait_scatter(2)
                issue_idx(c0 + 2, 2)

            wait_gather(1)
            wait_idx(c0 + 2, 2)
            issue_gather(2)
            scale(1)
            issue_scatter(1)
            wait_scatter(0)
            issue_idx(c0 + 3, 0)

            wait_gather(2)
            wait_idx(c0 + 3, 0)
            issue_gather(0)
            scale(2)
            issue_scatter(2)
            wait_scatter(1)
            issue_idx(c0 + 4, 1)
            return 0
        lax.fori_loop(0, n_triples - 1, triple_body, 0)

        c0 = cbase + 3 * (n_triples - 1)
        wait_gather(0)
        wait_idx(c0 + 1, 1)
        issue_gather(1)
        scale(0)
        issue_scatter(0)
        wait_scatter(2)
        issue_idx(c0 + 2, 2)

        wait_gather(1)
        wait_idx(c0 + 2, 2)
        issue_gather(2)
        scale(1)
        issue_scatter(1)

        wait_gather(2)
        scale(2)
        issue_scatter(2)

        wait_scatter(0)
        wait_scatter(1)
        wait_scatter(2)
        plsc.subcore_barrier()

        def rb_body(k, _):
            r0 = s * ROWS_PER_TILE + k * RB
            pltpu.sync_copy(acc.at[pl.ds(r0, RB)], rows[0].at[pl.ds(0, RB)])
            pltpu.sync_copy(rows[0].at[pl.ds(0, RB)],
                            out_hbm.at[c, pl.ds(r0, RB)])
            return 0
        lax.fori_loop(0, ROWS_PER_TILE // RB, rb_body, 0)

    return spmm_kernel(x, packed, wblk)


_BR = 1024


def _dense_relu_tc(y, w, b):
    def body(y_ref, w_ref, b_ref, o_ref):
        ysum = y_ref[0] + y_ref[1]
        acc = lax.dot_general(ysum, w_ref[...], (((1,), (1,)), ((), ())),
                              preferred_element_type=jnp.float32)
        o_ref[...] = jnp.maximum(acc + b_ref[...], 0.0)

    return pl.pallas_call(
        body,
        grid=(N_PAD // _BR,),
        in_specs=[
            pl.BlockSpec((NC, _BR, DIM), lambda i: (0, i, 0)),
            pl.BlockSpec((DIM, DIM), lambda i: (0, 0)),
            pl.BlockSpec((1, DIM), lambda i: (0, 0)),
        ],
        out_specs=pl.BlockSpec((_BR, DIM), lambda i: (i, 0)),
        out_shape=jax.ShapeDtypeStruct((N_PAD, DIM), jnp.float32),
    )(y, w, b.reshape(1, DIM))


def _final_tc(z, idx3, w2, b2, w3, b3, w4, b4):
    nb = N_PAD // _BR

    def body(z_ref, idx_ref, w2_ref, b2_ref, w3_ref, b3_ref, w4_ref, b4_ref,
             o_ref, pool_ref):
        i = pl.program_id(0)

        @pl.when(i == 0)
        def _():
            pool_ref[...] = jnp.zeros((NUM_GRAPHS, DIM), jnp.float32)

        zsum = z_ref[0] + z_ref[1]
        x2 = lax.dot_general(zsum, w2_ref[...], (((1,), (1,)), ((), ())),
                             preferred_element_type=jnp.float32)
        x2 = jnp.maximum(x2 + b2_ref[...], 0.0)

        gids = lax.broadcasted_iota(jnp.int32, (NUM_GRAPHS, _BR), 0)
        onehot = (gids == idx_ref[0]).astype(jnp.float32)
        pool_ref[...] += lax.dot_general(
            onehot, x2, (((1,), (0,)), ((), ())),
            preferred_element_type=jnp.float32)

        @pl.when(i == nb - 1)
        def _():
            h = lax.dot_general(pool_ref[...], w3_ref[...],
                                (((1,), (1,)), ((), ())),
                                preferred_element_type=jnp.float32)
            h = jnp.maximum(h + b3_ref[...], 0.0)
            logits = lax.dot_general(h, w4_ref[...], (((1,), (1,)), ((), ())),
                                     preferred_element_type=jnp.float32)
            logits = logits + b4_ref[...]
            m = jnp.max(logits, axis=1, keepdims=True)
            lse = m + jnp.log(jnp.sum(jnp.exp(logits - m), axis=1,
                                      keepdims=True))
            o_ref[...] = logits - lse

    return pl.pallas_call(
        body,
        grid=(nb,),
        in_specs=[
            pl.BlockSpec((NC, _BR, DIM), lambda i: (0, i, 0)),
            pl.BlockSpec((1, 1, _BR), lambda i: (i, 0, 0)),
            pl.BlockSpec((DIM, DIM), lambda i: (0, 0)),
            pl.BlockSpec((1, DIM), lambda i: (0, 0)),
            pl.BlockSpec((DIM, DIM), lambda i: (0, 0)),
            pl.BlockSpec((1, DIM), lambda i: (0, 0)),
            pl.BlockSpec((NUM_CLASSES, DIM), lambda i: (0, 0)),
            pl.BlockSpec((1, NUM_CLASSES), lambda i: (0, 0)),
        ],
        out_specs=pl.BlockSpec((NUM_GRAPHS, NUM_CLASSES), lambda i: (0, 0)),
        out_shape=jax.ShapeDtypeStruct((NUM_GRAPHS, NUM_CLASSES), jnp.float32),
        scratch_shapes=[pltpu.VMEM((NUM_GRAPHS, DIM), jnp.float32)],
    )(z, idx3, w2, b2.reshape(1, DIM), w3, b3.reshape(1, DIM), w4,
      b4.reshape(1, NUM_CLASSES))


def kernel(x_in, edge_index, edge_weight, idx, W1, b1, W2, b2, W3, b3, W4, b4):
    e = edge_index.shape[1]
    pad = E_PAD - e
    row = edge_index[0].astype(jnp.int32)
    col = edge_index[1].astype(jnp.int32)
    if pad:
        row = jnp.concatenate([row, jnp.zeros((pad,), jnp.int32)])
        col = jnp.concatenate([col, jnp.zeros((pad,), jnp.int32)])
        w = jnp.concatenate([edge_weight, jnp.zeros((pad,), jnp.float32)])
    else:
        w = edge_weight
    packed = jnp.stack([col.reshape(-1, CHUNK), row.reshape(-1, CHUNK)],
                       axis=1)
    wblk = w.reshape(-1, CHUNK)

    y = _spmm_sc(x_in, packed, wblk)
    x1 = _dense_relu_tc(y, W1, b1)
    z = _spmm_sc(x1, packed, wblk)
    idxp = jnp.concatenate([idx.astype(jnp.int32),
                            jnp.full((N_PAD - N_NODES,), NUM_GRAPHS, jnp.int32)])
    idx3 = idxp.reshape(N_PAD // _BR, 1, _BR)
    return _final_tc(z, idx3, W2, b2, W3, b3, W4, b4)
